# Initial kernel scaffold; baseline (speedup 1.0000x reference)
import jax
import jax.numpy as jnp
from jax.experimental import pallas as pl


def kernel(in_tensor, w0, w1, fov0, fov1):
    raise NotImplementedError("write your pallas kernel here")


# speedup vs baseline: 6.7879x; 6.7879x over previous
import functools

import jax
import jax.numpy as jnp
from jax.experimental import pallas as pl
from jax.experimental.pallas import tpu as pltpu


def _resize_body(x_ref, a0_ref, a1t_ref, o_ref):
    x = x_ref[0]
    y = jnp.dot(a0_ref[...], x, preferred_element_type=jnp.float32)
    o_ref[0] = jnp.dot(y, a1t_ref[...], preferred_element_type=jnp.float32)


@functools.partial(jax.jit, static_argnames=("interpret",))
def _resize(x, a0, a1t, interpret=False):
    n, h, w = x.shape
    oh = a0.shape[0]
    ow = a1t.shape[1]
    return pl.pallas_call(
        _resize_body,
        grid=(n,),
        in_specs=[
            pl.BlockSpec((1, h, w), lambda i: (i, 0, 0)),
            pl.BlockSpec((oh, h), lambda i: (0, 0)),
            pl.BlockSpec((w, ow), lambda i: (0, 0)),
        ],
        out_specs=pl.BlockSpec((1, oh, ow), lambda i: (i, 0, 0)),
        out_shape=jax.ShapeDtypeStruct((n, oh, ow), jnp.float32),
        compiler_params=pltpu.CompilerParams(
            dimension_semantics=("parallel",),
        ),
        interpret=interpret,
    )(x, a0, a1t)


def _dense_resize_matrix(fov, w, in_length):
    taps, out = fov.shape
    cols = fov.astype(jnp.int32)
    rows = jnp.broadcast_to(jnp.arange(out, dtype=jnp.int32)[None, :], (taps, out))
    a = jnp.zeros((out, in_length), jnp.float32)
    return a.at[rows, cols].add(w.astype(jnp.float32))


def kernel(in_tensor, w0, w1, fov0, fov1, interpret=False):
    b, c, h, w = in_tensor.shape
    taps, oh = fov0.shape
    ow = fov1.shape[1]
    a0 = _dense_resize_matrix(fov0, w0.reshape(taps, oh), h)
    a1t = _dense_resize_matrix(fov1, w1.reshape(taps, ow), w).T
    x = in_tensor.reshape(b * c, h, w)
    out = _resize(x, a0, a1t, interpret=interpret)
    return out.reshape(b, c, oh, ow)


# speedup vs baseline: 12.4109x; 1.8284x over previous
import functools

import jax
import jax.numpy as jnp
from jax.experimental import pallas as pl
from jax.experimental.pallas import tpu as pltpu


def _densify_body(fov0t_ref, w0t_ref, fov1_ref, w1_ref, a0_ref, a1t_ref):
    oh, taps = fov0t_ref.shape
    h = a0_ref.shape[1]
    w_in, ow = a1t_ref.shape
    col = jax.lax.broadcasted_iota(jnp.int32, (oh, h), 1)
    acc0 = jnp.zeros((oh, h), jnp.float32)
    for p in range(taps):
        acc0 += jnp.where(fov0t_ref[:, p : p + 1] == col,
                          w0t_ref[:, p : p + 1], 0.0)
    a0_ref[...] = acc0
    row = jax.lax.broadcasted_iota(jnp.int32, (w_in, ow), 0)
    acc1 = jnp.zeros((w_in, ow), jnp.float32)
    for p in range(taps):
        acc1 += jnp.where(fov1_ref[p : p + 1, :] == row,
                          w1_ref[p : p + 1, :], 0.0)
    a1t_ref[...] = acc1


def _resize_body(x_ref, a0_ref, a1t_ref, o_ref):
    x = x_ref[0]
    y = jnp.dot(a0_ref[...], x, preferred_element_type=jnp.float32)
    o_ref[0] = jnp.dot(y, a1t_ref[...], preferred_element_type=jnp.float32)


@functools.partial(jax.jit, static_argnames=("h", "w", "interpret"))
def _densify(fov0t, w0t, fov1, w1, h, w, interpret=False):
    oh = fov0t.shape[0]
    ow = fov1.shape[1]
    return pl.pallas_call(
        _densify_body,
        out_shape=(
            jax.ShapeDtypeStruct((oh, h), jnp.float32),
            jax.ShapeDtypeStruct((w, ow), jnp.float32),
        ),
        interpret=interpret,
    )(fov0t, w0t, fov1, w1)


@functools.partial(jax.jit, static_argnames=("interpret",))
def _resize(x, a0, a1t, interpret=False):
    n, h, w = x.shape
    oh = a0.shape[0]
    ow = a1t.shape[1]
    return pl.pallas_call(
        _resize_body,
        grid=(n,),
        in_specs=[
            pl.BlockSpec((1, h, w), lambda i: (i, 0, 0)),
            pl.BlockSpec((oh, h), lambda i: (0, 0)),
            pl.BlockSpec((w, ow), lambda i: (0, 0)),
        ],
        out_specs=pl.BlockSpec((1, oh, ow), lambda i: (i, 0, 0)),
        out_shape=jax.ShapeDtypeStruct((n, oh, ow), jnp.float32),
        compiler_params=pltpu.CompilerParams(
            dimension_semantics=("parallel",),
        ),
        interpret=interpret,
    )(x, a0, a1t)


def kernel(in_tensor, w0, w1, fov0, fov1, interpret=False):
    b, c, h, w = in_tensor.shape
    taps, oh = fov0.shape
    ow = fov1.shape[1]
    fov0t = fov0.astype(jnp.int32).T
    w0t = w0.reshape(taps, oh).astype(jnp.float32).T
    a0, a1t = _densify(fov0t, w0t, fov1.astype(jnp.int32),
                       w1.reshape(taps, ow).astype(jnp.float32),
                       h, w, interpret=interpret)
    x = in_tensor.reshape(b * c, h, w)
    out = _resize(x, a0, a1t, interpret=interpret)
    return out.reshape(b, c, oh, ow)
